# Initial kernel scaffold; baseline (speedup 1.0000x reference)
import functools

import jax
import jax.numpy as jnp
from jax import lax
from jax.experimental import pallas as pl
from jax.experimental.pallas import tpu as pltpu
from jax.experimental.pallas import tpu_sc as plsc

NN = 10000
EE = 320000
HH = 8
DD = 16
HD = HH * DD
CLAMP = 5.0

NW = 32
EPW = EE // NW
CB = 80
NCHUNK = EPW // CB
ROWS_PER_SUB = NN // 16


def _linear_body(x_ref, w_ref, b_ref, o_ref):
    o_ref[...] = (
        jnp.dot(x_ref[...], w_ref[...], preferred_element_type=jnp.float32)
        + b_ref[...]
    )


def _linear(x, w, b, rb):
    m, k = x.shape
    n = w.shape[1]
    return pl.pallas_call(
        _linear_body,
        grid=(m // rb,),
        in_specs=[
            pl.BlockSpec((rb, k), lambda i: (i, 0)),
            pl.BlockSpec((k, n), lambda i: (0, 0)),
            pl.BlockSpec((1, n), lambda i: (0, 0)),
        ],
        out_specs=pl.BlockSpec((rb, n), lambda i: (i, 0)),
        out_shape=jax.ShapeDtypeStruct((m, n), jnp.float32),
    )(x, w, b.reshape(1, -1))


def _edgew_body(c_ref, ewbd_ref, awm16_ref, awm8_ref, tw_ref, w8_ref):
    c = c_ref[...]
    t = jnp.dot(c, ewbd_ref[...], preferred_element_type=jnp.float32)
    w16 = jnp.exp(jnp.clip(
        jnp.dot(c, awm16_ref[...], preferred_element_type=jnp.float32),
        -CLAMP, CLAMP))
    tw_ref[...] = w16 * t
    w8_ref[...] = jnp.exp(jnp.clip(
        jnp.dot(c, awm8_ref[...], preferred_element_type=jnp.float32),
        -CLAMP, CLAMP))


def _edge_weights(conn, ewbd, awm16, awm8, rb):
    return pl.pallas_call(
        _edgew_body,
        grid=(EE // rb,),
        in_specs=[
            pl.BlockSpec((rb, HD), lambda i: (i, 0)),
            pl.BlockSpec((HD, HD), lambda i: (0, 0)),
            pl.BlockSpec((HD, HD), lambda i: (0, 0)),
            pl.BlockSpec((HD, HH), lambda i: (0, 0)),
        ],
        out_specs=[
            pl.BlockSpec((rb, HD), lambda i: (i, 0)),
            pl.BlockSpec((rb, HH), lambda i: (i, 0)),
        ],
        out_shape=[
            jax.ShapeDtypeStruct((EE, HD), jnp.float32),
            jax.ShapeDtypeStruct((EE, HH), jnp.float32),
        ],
    )(conn, ewbd, awm16, awm8)


def _final_body(agg_ref, den_ref, r8_ref, o_ref):
    agg = agg_ref[0] + agg_ref[1]
    den = den_ref[0] + den_ref[1]
    d16 = jnp.dot(den, r8_ref[...], preferred_element_type=jnp.float32)
    o_ref[...] = agg / (d16 + 1e-16)


def _finalize(agg_p, den_p, r8, rb):
    return pl.pallas_call(
        _final_body,
        grid=(NN // rb,),
        in_specs=[
            pl.BlockSpec((2, rb, HD), lambda i: (0, i, 0)),
            pl.BlockSpec((2, rb, HH), lambda i: (0, i, 0)),
            pl.BlockSpec((HH, HD), lambda i: (0, 0)),
        ],
        out_specs=pl.BlockSpec((rb, HD), lambda i: (i, 0)),
        out_shape=jax.ShapeDtypeStruct((NN, HD), jnp.float32),
    )(agg_p, den_p, r8)


_MESH = plsc.VectorSubcoreMesh(core_axis_name="c", subcore_axis_name="s")


@functools.partial(
    pl.kernel,
    mesh=_MESH,
    out_type=jax.ShapeDtypeStruct((EE, HD), jnp.float32),
    scratch_types=[
        pltpu.VMEM((CB,), jnp.int32),
        pltpu.VMEM((CB,), jnp.int32),
        pltpu.VMEM((CB, HD), jnp.float32),
        pltpu.VMEM((CB, HD), jnp.float32),
        pltpu.VMEM((CB, HD), jnp.float32),
        pltpu.SemaphoreType.DMA,
        pltpu.SemaphoreType.DMA,
        pltpu.SemaphoreType.DMA,
    ],
)
def _conn_sc(nk_hbm, nq_hbm, eq_hbm, src_hbm, dst_hbm, out_hbm,
             src_v, dst_v, nk_v, nq_v, cn_v, sem1, sem2, sem3):
    wid = lax.axis_index("s") * 2 + lax.axis_index("c")

    def chunk(i, _):
        base = wid * EPW + i * CB
        pltpu.sync_copy(src_hbm.at[pl.ds(base, CB)], src_v)
        pltpu.sync_copy(dst_hbm.at[pl.ds(base, CB)], dst_v)
        cp1 = pltpu.async_copy(nk_hbm.at[src_v], nk_v, sem1)
        cp2 = pltpu.async_copy(nq_hbm.at[dst_v], nq_v, sem2)
        cp3 = pltpu.async_copy(eq_hbm.at[pl.ds(base, CB)], cn_v, sem3)
        cp1.wait()
        cp2.wait()
        cp3.wait()

        def edge(e, _):
            for h in range(HH):
                sl = (e, pl.ds(h * DD, DD))
                cn_v[sl] = jnp.maximum(nk_v[sl] + nq_v[sl] + cn_v[sl], 0.0)
            return 0

        lax.fori_loop(0, CB, edge, 0)
        pltpu.sync_copy(cn_v, out_hbm.at[pl.ds(base, CB)])
        return 0

    lax.fori_loop(0, NCHUNK, chunk, 0)


@functools.partial(
    pl.kernel,
    mesh=_MESH,
    out_type=(
        jax.ShapeDtypeStruct((2, NN, HD), jnp.float32),
        jax.ShapeDtypeStruct((2, NN, HH), jnp.float32),
    ),
    scratch_types=[
        pltpu.VMEM((CB,), jnp.int32),
        pltpu.VMEM((CB,), jnp.int32),
        pltpu.VMEM((CB, HD), jnp.float32),
        pltpu.VMEM((CB, HD), jnp.float32),
        pltpu.VMEM((CB, HH), jnp.float32),
        pltpu.VMEM_SHARED((NN, HD), jnp.float32),
        pltpu.VMEM_SHARED((NN, HH), jnp.float32),
        pltpu.SemaphoreType.DMA,
    ],
)
def _agg_sc(tw_hbm, w8_hbm, nv_hbm, src_hbm, dst_hbm, z128_hbm, z8_hbm,
            agg_out, den_out,
            src_v, dst_v, tw_v, nv_v, w8_v, agg_sh, den_sh, sem):
    cid = lax.axis_index("c")
    sid = lax.axis_index("s")
    wid = sid * 2 + cid
    row0 = sid * ROWS_PER_SUB

    pltpu.sync_copy(z128_hbm.at[pl.ds(row0, ROWS_PER_SUB)],
                    agg_sh.at[pl.ds(row0, ROWS_PER_SUB)])
    pltpu.sync_copy(z8_hbm.at[pl.ds(row0, ROWS_PER_SUB)],
                    den_sh.at[pl.ds(row0, ROWS_PER_SUB)])
    plsc.subcore_barrier()

    def chunk(i, _):
        base = wid * EPW + i * CB
        pltpu.sync_copy(src_hbm.at[pl.ds(base, CB)], src_v)
        pltpu.sync_copy(dst_hbm.at[pl.ds(base, CB)], dst_v)
        cp = pltpu.async_copy(nv_hbm.at[src_v], nv_v, sem)
        pltpu.sync_copy(tw_hbm.at[pl.ds(base, CB)], tw_v)
        pltpu.sync_copy(w8_hbm.at[pl.ds(base, CB)], w8_v)
        cp.wait()

        def edge(e, _):
            for h in range(HH):
                wv = w8_v[e, h]
                sl = (e, pl.ds(h * DD, DD))
                tw_v[sl] = tw_v[sl] + wv * nv_v[sl]
            return 0

        lax.fori_loop(0, CB, edge, 0)
        pltpu.sync_copy(tw_v, agg_sh.at[dst_v], add=True)
        pltpu.sync_copy(w8_v, den_sh.at[dst_v], add=True)
        return 0

    lax.fori_loop(0, NCHUNK, chunk, 0)
    plsc.subcore_barrier()

    pltpu.sync_copy(agg_sh.at[pl.ds(row0, ROWS_PER_SUB)],
                    agg_out.at[cid, pl.ds(row0, ROWS_PER_SUB)])
    pltpu.sync_copy(den_sh.at[pl.ds(row0, ROWS_PER_SUB)],
                    den_out.at[cid, pl.ds(row0, ROWS_PER_SUB)])


def kernel(x, edge_index, edge_attr, Wq, bq, Wk, bk, Wv, bv, We, be, Aw, Ew):
    src = edge_index[0]
    dst = edge_index[1]

    awm16 = jnp.zeros((HD, HD), jnp.float32)
    ewbd = jnp.zeros((HD, HD), jnp.float32)
    awm8 = jnp.zeros((HD, HH), jnp.float32)
    r8 = jnp.zeros((HH, HD), jnp.float32)
    for h in range(HH):
        blk = Aw[:, h, 0:1] * jnp.ones((1, DD), jnp.float32)
        awm16 = awm16.at[h * DD:(h + 1) * DD, h * DD:(h + 1) * DD].set(blk)
        ewbd = ewbd.at[h * DD:(h + 1) * DD, h * DD:(h + 1) * DD].set(Ew[:, h, :])
        awm8 = awm8.at[h * DD:(h + 1) * DD, h].set(Aw[:, h, 0])
        r8 = r8.at[h, h * DD:(h + 1) * DD].set(1.0)

    wqkv = jnp.concatenate([Wq, Wk, Wv], axis=1)
    bqkv = jnp.concatenate([bq, bk, bv], axis=0)

    nqkv = _linear(x, wqkv, bqkv, rb=2000)
    nq = nqkv[:, :HD]
    nk = nqkv[:, HD:2 * HD]
    nv = nqkv[:, 2 * HD:]
    eq = _linear(edge_attr, We, be, rb=2000)

    conn = _conn_sc(nk, nq, eq, src, dst)

    tw, w8 = _edge_weights(conn, ewbd, awm16, awm8, rb=2000)

    z128 = jnp.zeros((NN, HD), jnp.float32)
    z8 = jnp.zeros((NN, HH), jnp.float32)
    agg_p, den_p = _agg_sc(tw, w8, nv, src, dst, z128, z8)

    n_out = _finalize(agg_p, den_p, r8, rb=2000)
    return (n_out, conn)


# speedup vs baseline: 40.9717x; 40.9717x over previous
import functools

import jax
import jax.numpy as jnp
from jax import lax
from jax.experimental import pallas as pl
from jax.experimental.pallas import tpu as pltpu
from jax.experimental.pallas import tpu_sc as plsc

NN = 10000
EE = 320000
HH = 8
DD = 16
HD = HH * DD
CLAMP = 5.0

NW = 32
EPW = EE // NW
CB = 80
NCHUNK = EPW // CB
NP = 10240
ROWS_PER_SUB = NP // 16
NPR = NP // 8
DRPS = NPR // 16


def _linear_body(x_ref, w_ref, b_ref, o_ref):
    o_ref[...] = (
        jnp.dot(x_ref[...], w_ref[...], preferred_element_type=jnp.float32)
        + b_ref[...]
    )


def _linear(x, w, b, rb):
    m, k = x.shape
    n = w.shape[1]
    return pl.pallas_call(
        _linear_body,
        grid=(m // rb,),
        in_specs=[
            pl.BlockSpec((rb, k), lambda i: (i, 0)),
            pl.BlockSpec((k, n), lambda i: (0, 0)),
            pl.BlockSpec((1, n), lambda i: (0, 0)),
        ],
        out_specs=pl.BlockSpec((rb, n), lambda i: (i, 0)),
        out_shape=jax.ShapeDtypeStruct((m, n), jnp.float32),
    )(x, w, b.reshape(1, -1))


def _edgew_body(c_ref, dst_ref, ewbd_ref, awm16_ref, awm8_ref, awm8t_ref,
                tw_ref, w8_ref, denr_ref):
    c = c_ref[...]
    t = jnp.dot(c, ewbd_ref[...], preferred_element_type=jnp.float32)
    w16 = jnp.exp(jnp.clip(
        jnp.dot(c, awm16_ref[...], preferred_element_type=jnp.float32),
        -CLAMP, CLAMP))
    tw_ref[...] = w16 * t
    w8_ref[...] = jnp.exp(jnp.clip(
        jnp.dot(c, awm8_ref[...], preferred_element_type=jnp.float32),
        -CLAMP, CLAMP))
    w128 = jnp.exp(jnp.clip(
        jnp.dot(c, awm8t_ref[...], preferred_element_type=jnp.float32),
        -CLAMP, CLAMP))
    lane = lax.broadcasted_iota(jnp.int32, w128.shape, 1)
    slot = lax.rem(dst_ref[...], jnp.int32(8))
    denr_ref[...] = jnp.where((lane >> 4) == slot, w128, 0.0)


def _edge_weights(conn, dst2d, ewbd, awm16, awm8, awm8t, rb):
    return pl.pallas_call(
        _edgew_body,
        grid=(EE // rb,),
        in_specs=[
            pl.BlockSpec((rb, HD), lambda i: (i, 0)),
            pl.BlockSpec((rb, 1), lambda i: (i, 0)),
            pl.BlockSpec((HD, HD), lambda i: (0, 0)),
            pl.BlockSpec((HD, HD), lambda i: (0, 0)),
            pl.BlockSpec((HD, 16), lambda i: (0, 0)),
            pl.BlockSpec((HD, HD), lambda i: (0, 0)),
        ],
        out_specs=[
            pl.BlockSpec((rb, HD), lambda i: (i, 0)),
            pl.BlockSpec((rb, 16), lambda i: (i, 0)),
            pl.BlockSpec((rb, HD), lambda i: (i, 0)),
        ],
        out_shape=[
            jax.ShapeDtypeStruct((EE, HD), jnp.float32),
            jax.ShapeDtypeStruct((EE, 16), jnp.float32),
            jax.ShapeDtypeStruct((EE, HD), jnp.float32),
        ],
    )(conn, dst2d, ewbd, awm16, awm8, awm8t)


def _final_body(agg_ref, den_ref, r8_ref, o_ref):
    agg = agg_ref[0] + agg_ref[1]
    den = den_ref[0] + den_ref[1]
    d16 = jnp.dot(den, r8_ref[...], preferred_element_type=jnp.float32)
    o_ref[...] = agg / (d16 + 1e-16)


def _finalize(agg_p, den_p, r8, rb):
    return pl.pallas_call(
        _final_body,
        grid=(NN // rb,),
        in_specs=[
            pl.BlockSpec((2, rb, HD), lambda i: (0, i, 0)),
            pl.BlockSpec((2, rb, 16), lambda i: (0, i, 0)),
            pl.BlockSpec((16, HD), lambda i: (0, 0)),
        ],
        out_specs=pl.BlockSpec((rb, HD), lambda i: (i, 0)),
        out_shape=jax.ShapeDtypeStruct((NN, HD), jnp.float32),
    )(agg_p, den_p, r8)


_MESH = plsc.VectorSubcoreMesh(core_axis_name="c", subcore_axis_name="s")


@functools.partial(
    pl.kernel,
    mesh=_MESH,
    out_type=jax.ShapeDtypeStruct((EE, HD), jnp.float32),
    scratch_types=[
        pltpu.VMEM((CB,), jnp.int32),
        pltpu.VMEM((CB,), jnp.int32),
        pltpu.VMEM((CB, HD), jnp.float32),
        pltpu.VMEM((CB, HD), jnp.float32),
        pltpu.VMEM((CB, HD), jnp.float32),
        pltpu.SemaphoreType.DMA,
        pltpu.SemaphoreType.DMA,
        pltpu.SemaphoreType.DMA,
    ],
)
def _conn_sc(nk_hbm, nq_hbm, eq_hbm, src_hbm, dst_hbm, out_hbm,
             src_v, dst_v, nk_v, nq_v, cn_v, sem1, sem2, sem3):
    wid = lax.axis_index("s") * 2 + lax.axis_index("c")

    def chunk(i, _):
        base = wid * EPW + i * CB
        pltpu.sync_copy(src_hbm.at[pl.ds(base, CB)], src_v)
        pltpu.sync_copy(dst_hbm.at[pl.ds(base, CB)], dst_v)
        cp1 = pltpu.async_copy(nk_hbm.at[src_v], nk_v, sem1)
        cp2 = pltpu.async_copy(nq_hbm.at[dst_v], nq_v, sem2)
        cp3 = pltpu.async_copy(eq_hbm.at[pl.ds(base, CB)], cn_v, sem3)
        cp1.wait()
        cp2.wait()
        cp3.wait()

        def edge(e, _):
            for h in range(HH):
                sl = (e, pl.ds(h * DD, DD))
                cn_v[sl] = jnp.maximum(nk_v[sl] + nq_v[sl] + cn_v[sl], 0.0)
            return 0

        lax.fori_loop(0, CB, edge, 0)
        pltpu.sync_copy(cn_v, out_hbm.at[pl.ds(base, CB)])
        return 0

    lax.fori_loop(0, NCHUNK, chunk, 0)


@functools.partial(
    pl.kernel,
    mesh=_MESH,
    out_type=(
        jax.ShapeDtypeStruct((2, NP, HD), jnp.float32),
        jax.ShapeDtypeStruct((2, NPR, HD), jnp.float32),
    ),
    scratch_types=[
        pltpu.VMEM((CB,), jnp.int32),
        pltpu.VMEM((CB,), jnp.int32),
        pltpu.VMEM((CB, HD), jnp.float32),
        pltpu.VMEM((CB, HD), jnp.float32),
        pltpu.VMEM((CB, HD), jnp.float32),
        pltpu.VMEM((CB * 16,), jnp.float32),
        pltpu.VMEM_SHARED((NP, HD), jnp.float32),
        pltpu.VMEM_SHARED((NPR, HD), jnp.float32),
        pltpu.SemaphoreType.DMA,
    ],
)
def _agg_sc(tw_hbm, w8f_hbm, denr_hbm, nv_hbm, src_hbm, dst_hbm, z128_hbm,
            agg_out, den_out,
            src_v, dst8_v, tw_v, nv_v, denr_v, w8f_v, agg_sh, den_sh, sem):
    cid = lax.axis_index("c")
    sid = lax.axis_index("s")
    wid = sid * 2 + cid
    row0 = pl.multiple_of(sid * ROWS_PER_SUB, ROWS_PER_SUB)
    drow0 = pl.multiple_of(sid * DRPS, DRPS)

    pltpu.sync_copy(z128_hbm, tw_v)
    for k in range(ROWS_PER_SUB // CB):
        pltpu.sync_copy(tw_v, agg_sh.at[pl.ds(row0 + k * CB, CB)])
    pltpu.sync_copy(tw_v, den_sh.at[pl.ds(drow0, DRPS)])
    plsc.subcore_barrier()

    def chunk(i, _):
        base = wid * EPW + i * CB
        pltpu.sync_copy(src_hbm.at[pl.ds(base, CB)], src_v)
        pltpu.sync_copy(dst_hbm.at[pl.ds(base, CB)], dst8_v)
        cp = pltpu.async_copy(nv_hbm.at[src_v], nv_v, sem)
        pltpu.sync_copy(tw_hbm.at[pl.ds(base, CB)], tw_v)
        pltpu.sync_copy(denr_hbm.at[pl.ds(base, CB)], denr_v)
        pltpu.sync_copy(w8f_hbm.at[pl.ds(base * 16, CB * 16)], w8f_v)

        def shift(kk, _):
            off = kk * 16
            dv = dst8_v[pl.ds(off, 16)]
            dst8_v[pl.ds(off, 16)] = dv >> 3
            return 0

        lax.fori_loop(0, CB // 16, shift, 0)
        cp.wait()

        def edge(e, _):
            wvec = w8f_v[pl.ds(e * 16, 16)]
            for h in range(HH):
                sl = (e, pl.ds(h * DD, DD))
                tw_v[sl] = tw_v[sl] + wvec[h] * nv_v[sl]
            return 0

        lax.fori_loop(0, CB, edge, 0)
        pltpu.sync_copy(denr_v, den_sh.at[dst8_v], add=True)

        pltpu.sync_copy(dst_hbm.at[pl.ds(base, CB)], src_v)
        pltpu.sync_copy(tw_v, agg_sh.at[src_v], add=True)
        return 0

    lax.fori_loop(0, NCHUNK, chunk, 0)
    plsc.subcore_barrier()

    for k in range(ROWS_PER_SUB // CB):
        r = row0 + k * CB
        pltpu.sync_copy(agg_sh.at[pl.ds(r, CB)], tw_v)
        pltpu.sync_copy(tw_v, agg_out.at[cid, pl.ds(r, CB)])
    pltpu.sync_copy(den_sh.at[pl.ds(drow0, DRPS)], denr_v)
    pltpu.sync_copy(denr_v, den_out.at[cid, pl.ds(drow0, DRPS)])


def kernel(x, edge_index, edge_attr, Wq, bq, Wk, bk, Wv, bv, We, be, Aw, Ew):
    src = edge_index[0]
    dst = edge_index[1]

    awm16 = jnp.zeros((HD, HD), jnp.float32)
    ewbd = jnp.zeros((HD, HD), jnp.float32)
    awm8 = jnp.zeros((HD, 16), jnp.float32)
    r8 = jnp.zeros((16, HD), jnp.float32)
    for h in range(HH):
        blk = Aw[:, h, 0:1] * jnp.ones((1, DD), jnp.float32)
        awm16 = awm16.at[h * DD:(h + 1) * DD, h * DD:(h + 1) * DD].set(blk)
        ewbd = ewbd.at[h * DD:(h + 1) * DD, h * DD:(h + 1) * DD].set(Ew[:, h, :])
        awm8 = awm8.at[h * DD:(h + 1) * DD, h].set(Aw[:, h, 0])
        awm8 = awm8.at[h * DD:(h + 1) * DD, h + 8].set(Aw[:, h, 0])
        r8 = r8.at[h, h * DD:(h + 1) * DD].set(1.0)

    wqkv = jnp.concatenate([Wq, Wk, Wv], axis=1)
    bqkv = jnp.concatenate([bq, bk, bv], axis=0)

    nqkv = _linear(x, wqkv, bqkv, rb=2000)
    nq = nqkv[:, :HD]
    nk = nqkv[:, HD:2 * HD]
    nv = nqkv[:, 2 * HD:]
    eq = _linear(edge_attr, We, be, rb=2000)

    conn = _conn_sc(nk, nq, eq, src, dst)

    awm8t = jnp.tile(awm8, (1, 8))
    tw, w8, denr = _edge_weights(conn, dst.reshape(-1, 1), ewbd,
                                 awm16, awm8, awm8t, rb=2000)

    z128 = jnp.zeros((CB, HD), jnp.float32)
    agg_p, denr_p = _agg_sc(tw, w8.reshape(-1), denr, nv, src, dst, z128)
    den_p = denr_p.reshape(2, NP, 16)

    n_out = _finalize(agg_p[:, :NN], den_p[:, :NN], r8, rb=2000)
    return (n_out, conn)
